# baseline (speedup 1.0000x reference)
import functools

import jax
import jax.numpy as jnp
from jax import lax
from jax.experimental import pallas as pl
from jax.experimental.pallas import tpu as pltpu
from jax.experimental.pallas import tpu_sc as plsc

B, AT, NBR, F = 8, 96, 16, 128
K = NBR - 1
NT = B * AT * NBR
NW = 32
TRIP_W = NT // NW
CH_T = 16
NCH = TRIP_W // CH_T


@functools.partial(
    pl.kernel,
    mesh=plsc.VectorSubcoreMesh(core_axis_name="c", subcore_axis_name="s"),
    out_type=jax.ShapeDtypeStruct((NT, NBR, F), jnp.float32),
    compiler_params=pltpu.CompilerParams(needs_layout_passes=False),
    scratch_types=[
        pltpu.VMEM((TRIP_W,), jnp.int32),
        pltpu.VMEM((CH_T, NBR, F), jnp.float32),
        pltpu.VMEM((CH_T, NBR, F), jnp.float32),
        pltpu.SemaphoreType.DMA,
        pltpu.SemaphoreType.DMA,
        pltpu.SemaphoreType.DMA,
        pltpu.SemaphoreType.DMA,
    ],
)
def _gather_kernel(table_hbm, nbr_hbm, out_hbm, idx_v, stage_a, stage_b,
                   gsem_a, gsem_b, wsem_a, wsem_b):
    cid = lax.axis_index("c")
    sid = lax.axis_index("s")
    wid = sid * 2 + cid
    trip_base = wid * TRIP_W
    mol = trip_base // (AT * NBR)

    pltpu.sync_copy(nbr_hbm.at[pl.ds(trip_base, TRIP_W)], idx_v)

    def build_vec(v, carry):
        idx_v[pl.ds(v * 16, 16)] = idx_v[pl.ds(v * 16, 16)] + mol * AT
        return carry

    lax.fori_loop(0, TRIP_W // 16, build_vec, 0)

    def g_start(c, stage, sem):
        pltpu.async_copy(table_hbm.at[idx_v.at[pl.ds(c * CH_T, CH_T)]], stage, sem)

    def g_wait(c, stage, sem):
        pltpu.make_async_copy(
            table_hbm.at[idx_v.at[pl.ds(c * CH_T, CH_T)]], stage, sem
        ).wait()

    def w_start(c, stage, sem):
        pltpu.async_copy(stage, out_hbm.at[pl.ds(trip_base + c * CH_T, CH_T)], sem)

    def w_drain(c, stage, sem):
        pltpu.make_async_copy(
            stage, out_hbm.at[pl.ds(trip_base + c * CH_T, CH_T)], sem
        ).wait()

    g_start(0, stage_a, gsem_a)
    g_start(1, stage_b, gsem_b)

    def pair_step(h, carry):
        c0 = h * 2
        g_wait(c0, stage_a, gsem_a)
        w_start(c0, stage_a, wsem_a)
        g_wait(c0 + 1, stage_b, gsem_b)
        w_start(c0 + 1, stage_b, wsem_b)
        w_drain(c0, stage_a, wsem_a)
        g_start(c0 + 2, stage_a, gsem_a)
        w_drain(c0 + 1, stage_b, wsem_b)
        g_start(c0 + 3, stage_b, gsem_b)
        return carry

    lax.fori_loop(0, NCH // 2 - 1, pair_step, 0)

    c_last = NCH - 2
    g_wait(c_last, stage_a, gsem_a)
    w_start(c_last, stage_a, wsem_a)
    g_wait(c_last + 1, stage_b, gsem_b)
    w_start(c_last + 1, stage_b, wsem_b)
    w_drain(c_last, stage_a, wsem_a)
    w_drain(c_last + 1, stage_b, wsem_b)


def kernel(edge_embedding, nbr_idx):
    table = edge_embedding.reshape(B * AT, NBR, F)
    nbr_flat = nbr_idx.reshape(NT)
    blocks = _gather_kernel(table, nbr_flat)
    blocks = blocks.reshape(B, AT, NBR, NBR, F)
    per_j = [
        jnp.concatenate(
            [blocks[:, :, j, :j], blocks[:, :, j, j + 1:]], axis=2
        )
        for j in range(NBR)
    ]
    return jnp.stack(per_j, axis=2)


# speedup vs baseline: 4.5512x; 4.5512x over previous
import functools

import jax
import jax.numpy as jnp
from jax import lax
from jax.experimental import pallas as pl
from jax.experimental.pallas import tpu as pltpu
from jax.experimental.pallas import tpu_sc as plsc

B, AT, NBR, F = 8, 96, 16, 128
K = NBR - 1
NT = B * AT * NBR
NOUT = NT * K
NW = 32
ROWS_W = NOUT // NW
ATOMS_W = ROWS_W // (K * NBR)
CHUNK = 128
NCH = ROWS_W // CHUNK


@functools.partial(
    pl.kernel,
    mesh=plsc.VectorSubcoreMesh(core_axis_name="c", subcore_axis_name="s"),
    out_type=jax.ShapeDtypeStruct((NOUT, F), jnp.float32),
    compiler_params=pltpu.CompilerParams(needs_layout_passes=False),
    scratch_types=[
        pltpu.VMEM((ATOMS_W * NBR,), jnp.int32),
        pltpu.VMEM((ROWS_W,), jnp.int32),
        pltpu.VMEM((CHUNK, F), jnp.float32),
        pltpu.VMEM((CHUNK, F), jnp.float32),
        pltpu.SemaphoreType.DMA,
        pltpu.SemaphoreType.DMA,
        pltpu.SemaphoreType.DMA,
        pltpu.SemaphoreType.DMA,
    ],
)
def _gather_kernel(table_hbm, nbr_hbm, out_hbm, nbr_v, idx_v, stage_a, stage_b,
                   gsem_a, gsem_b, wsem_a, wsem_b):
    cid = lax.axis_index("c")
    sid = lax.axis_index("s")
    wid = sid * 2 + cid
    atom_base = wid * ATOMS_W
    row_base = wid * ROWS_W
    mol = atom_base // AT

    pltpu.sync_copy(nbr_hbm.at[pl.ds(atom_base * NBR, ATOMS_W * NBR)], nbr_v)

    iota = lax.iota(jnp.int32, 16)
    kcols = [s - lax.shift_right_arithmetic(iota - (s + 1), 31) for s in range(K)]

    def build_atom(m, carry):
        base_vec = (nbr_v[pl.ds(m * NBR, NBR)] + mol * AT) * NBR
        for s in range(K):
            idx_v[pl.ds((m * K + s) * NBR, NBR)] = base_vec + kcols[s]
        return carry

    lax.fori_loop(0, ATOMS_W, build_atom, 0)

    def g_start(c, stage, sem):
        pltpu.async_copy(table_hbm.at[idx_v.at[pl.ds(c * CHUNK, CHUNK)]], stage, sem)

    def g_wait(c, stage, sem):
        pltpu.make_async_copy(
            table_hbm.at[idx_v.at[pl.ds(c * CHUNK, CHUNK)]], stage, sem
        ).wait()

    def w_start(c, stage, sem):
        pltpu.async_copy(stage, out_hbm.at[pl.ds(row_base + c * CHUNK, CHUNK)], sem)

    def w_drain(c, stage, sem):
        pltpu.make_async_copy(
            stage, out_hbm.at[pl.ds(row_base + c * CHUNK, CHUNK)], sem
        ).wait()

    g_start(0, stage_a, gsem_a)
    g_start(1, stage_b, gsem_b)

    def pair_step(h, carry):
        c0 = h * 2
        g_wait(c0, stage_a, gsem_a)
        w_start(c0, stage_a, wsem_a)
        g_wait(c0 + 1, stage_b, gsem_b)
        w_start(c0 + 1, stage_b, wsem_b)
        w_drain(c0, stage_a, wsem_a)
        g_start(c0 + 2, stage_a, gsem_a)
        w_drain(c0 + 1, stage_b, wsem_b)

        @pl.when(c0 + 3 < NCH)
        def _():
            g_start(c0 + 3, stage_b, gsem_b)

        return carry

    lax.fori_loop(0, (NCH - 1) // 2, pair_step, 0)

    c_last = NCH - 1
    g_wait(c_last, stage_a, gsem_a)
    w_start(c_last, stage_a, wsem_a)
    w_drain(c_last, stage_a, wsem_a)


def kernel(edge_embedding, nbr_idx):
    table = edge_embedding.reshape(NT, F)
    nbr_flat = nbr_idx.reshape(NT)
    out = _gather_kernel(table, nbr_flat)
    return out.reshape(B, AT, K, NBR, F).transpose(0, 1, 3, 2, 4)


# speedup vs baseline: 4.9182x; 1.0807x over previous
import functools

import jax
import jax.numpy as jnp
from jax import lax
from jax.experimental import pallas as pl
from jax.experimental.pallas import tpu as pltpu
from jax.experimental.pallas import tpu_sc as plsc

B, AT, NBR, F = 8, 96, 16, 128
K = NBR - 1
NT = B * AT * NBR
NOUT = NT * K
NW = 32
ROWS_W = NOUT // NW
ATOMS_W = ROWS_W // (K * NBR)
CHUNK = 128
NCH = ROWS_W // CHUNK


@functools.partial(
    pl.kernel,
    mesh=plsc.VectorSubcoreMesh(core_axis_name="c", subcore_axis_name="s"),
    out_type=jax.ShapeDtypeStruct((NOUT, F), jnp.float32),
    compiler_params=pltpu.CompilerParams(needs_layout_passes=False),
    scratch_types=[
        pltpu.VMEM((ATOMS_W * NBR,), jnp.int32),
        pltpu.VMEM((ROWS_W,), jnp.int32),
        pltpu.VMEM((CHUNK, F), jnp.float32),
        pltpu.VMEM((CHUNK, F), jnp.float32),
        pltpu.VMEM((CHUNK, F), jnp.float32),
        pltpu.VMEM((CHUNK, F), jnp.float32),
        pltpu.SemaphoreType.DMA,
        pltpu.SemaphoreType.DMA,
        pltpu.SemaphoreType.DMA,
        pltpu.SemaphoreType.DMA,
        pltpu.SemaphoreType.DMA,
        pltpu.SemaphoreType.DMA,
        pltpu.SemaphoreType.DMA,
        pltpu.SemaphoreType.DMA,
    ],
)
def _gather_kernel(table_hbm, nbr_hbm, out_hbm, nbr_v, idx_v,
                   stage_0, stage_1, stage_2, stage_3,
                   gsem_0, gsem_1, gsem_2, gsem_3,
                   wsem_0, wsem_1, wsem_2, wsem_3):
    stages = [stage_0, stage_1, stage_2, stage_3]
    gsems = [gsem_0, gsem_1, gsem_2, gsem_3]
    wsems = [wsem_0, wsem_1, wsem_2, wsem_3]
    cid = lax.axis_index("c")
    sid = lax.axis_index("s")
    wid = sid * 2 + cid
    atom_base = wid * ATOMS_W
    row_base = wid * ROWS_W
    mol = atom_base // AT

    pltpu.sync_copy(nbr_hbm.at[pl.ds(atom_base * NBR, ATOMS_W * NBR)], nbr_v)

    iota = lax.iota(jnp.int32, 16)
    kcols = [s - lax.shift_right_arithmetic(iota - (s + 1), 31) for s in range(K)]

    def build_atom(m, carry):
        base_vec = (nbr_v[pl.ds(m * NBR, NBR)] + mol * AT) * NBR
        for s in range(K):
            idx_v[pl.ds((m * K + s) * NBR, NBR)] = base_vec + kcols[s]
        return carry

    lax.fori_loop(0, ATOMS_W, build_atom, 0)

    def g_start(c, stage, sem):
        pltpu.async_copy(table_hbm.at[idx_v.at[pl.ds(c * CHUNK, CHUNK)]], stage, sem)

    def g_wait(c, stage, sem):
        pltpu.make_async_copy(
            table_hbm.at[idx_v.at[pl.ds(c * CHUNK, CHUNK)]], stage, sem
        ).wait()

    def w_start(c, stage, sem):
        pltpu.async_copy(stage, out_hbm.at[pl.ds(row_base + c * CHUNK, CHUNK)], sem)

    def w_drain(c, stage, sem):
        pltpu.make_async_copy(
            stage, out_hbm.at[pl.ds(row_base + c * CHUNK, CHUNK)], sem
        ).wait()

    g_start(0, stages[0], gsems[0])
    g_start(1, stages[1], gsems[1])

    def ring_step(h, carry):
        for k in range(4):
            c = h * 4 + k
            kn = (k + 2) % 4

            @pl.when(c < NCH)
            def _(c=c, k=k, kn=kn):
                @pl.when(c >= 2)
                def _():
                    w_drain(c - 2, stages[kn], wsems[kn])

                @pl.when(c + 2 < NCH)
                def _():
                    g_start(c + 2, stages[kn], gsems[kn])

                g_wait(c, stages[k], gsems[k])
                w_start(c, stages[k], wsems[k])

        return carry

    lax.fori_loop(0, (NCH + 3) // 4, ring_step, 0)

    w_drain(NCH - 2, stages[(NCH - 2) % 4], wsems[(NCH - 2) % 4])
    w_drain(NCH - 1, stages[(NCH - 1) % 4], wsems[(NCH - 1) % 4])


def kernel(edge_embedding, nbr_idx):
    table = edge_embedding.reshape(NT, F)
    nbr_flat = nbr_idx.reshape(NT)
    out = _gather_kernel(table, nbr_flat)
    return out.reshape(B, AT, K, NBR, F).transpose(0, 1, 3, 2, 4)


# speedup vs baseline: 6.3994x; 1.3012x over previous
import functools

import jax
import jax.numpy as jnp
from jax import lax
from jax.experimental import pallas as pl
from jax.experimental.pallas import tpu as pltpu
from jax.experimental.pallas import tpu_sc as plsc

B, AT, NBR, F = 8, 96, 16, 128
K = NBR - 1
NT = B * AT * NBR
NOUT = NT * K
NW = 32
ROWS_W = NOUT // NW
ATOMS_W = ROWS_W // (K * NBR)
CHUNK = 128
NCH = ROWS_W // CHUNK


@functools.partial(
    pl.kernel,
    mesh=plsc.VectorSubcoreMesh(core_axis_name="c", subcore_axis_name="s"),
    out_type=jax.ShapeDtypeStruct((NOUT, F), jnp.float32),
    compiler_params=pltpu.CompilerParams(needs_layout_passes=False),
    scratch_types=[
        pltpu.VMEM_SHARED((NT // 2, F), jnp.float32),
        pltpu.VMEM((ATOMS_W * NBR,), jnp.int32),
        pltpu.VMEM((ROWS_W,), jnp.int32),
        pltpu.VMEM((CHUNK, F), jnp.float32),
        pltpu.VMEM((CHUNK, F), jnp.float32),
        pltpu.VMEM((CHUNK, F), jnp.float32),
        pltpu.VMEM((CHUNK, F), jnp.float32),
        pltpu.SemaphoreType.DMA,
        pltpu.SemaphoreType.DMA,
        pltpu.SemaphoreType.DMA,
        pltpu.SemaphoreType.DMA,
        pltpu.SemaphoreType.DMA,
        pltpu.SemaphoreType.DMA,
        pltpu.SemaphoreType.DMA,
        pltpu.SemaphoreType.DMA,
    ],
)
def _gather_kernel(table_hbm, nbr_hbm, out_hbm, slab, nbr_v, idx_v,
                   stage_0, stage_1, stage_2, stage_3,
                   gsem_0, gsem_1, gsem_2, gsem_3,
                   wsem_0, wsem_1, wsem_2, wsem_3):
    stages = [stage_0, stage_1, stage_2, stage_3]
    gsems = [gsem_0, gsem_1, gsem_2, gsem_3]
    wsems = [wsem_0, wsem_1, wsem_2, wsem_3]
    cid = lax.axis_index("c")
    sid = lax.axis_index("s")
    wid = cid * 16 + sid
    atom_base = wid * ATOMS_W
    row_base = wid * ROWS_W
    mol = atom_base // AT

    slab_rows = NT // 2 // 16
    pltpu.sync_copy(
        table_hbm.at[pl.ds(cid * (NT // 2) + sid * slab_rows, slab_rows)],
        slab.at[pl.ds(sid * slab_rows, slab_rows)],
    )
    pltpu.sync_copy(nbr_hbm.at[pl.ds(atom_base * NBR, ATOMS_W * NBR)], nbr_v)
    plsc.subcore_barrier()

    iota = lax.iota(jnp.int32, 16)
    kcols = [s - lax.shift_right_arithmetic(iota - (s + 1), 31) for s in range(K)]

    mol_loc = mol - cid * (B // 2)

    def build_atom(m, carry):
        base_vec = (nbr_v[pl.ds(m * NBR, NBR)] + mol_loc * AT) * NBR
        for s in range(K):
            idx_v[pl.ds((m * K + s) * NBR, NBR)] = base_vec + kcols[s]
        return carry

    lax.fori_loop(0, ATOMS_W, build_atom, 0)

    def g_start(c, stage, sem):
        pltpu.async_copy(slab.at[idx_v.at[pl.ds(c * CHUNK, CHUNK)]], stage, sem)

    def g_wait(c, stage, sem):
        pltpu.make_async_copy(
            slab.at[idx_v.at[pl.ds(c * CHUNK, CHUNK)]], stage, sem
        ).wait()

    def w_start(c, stage, sem):
        pltpu.async_copy(stage, out_hbm.at[pl.ds(row_base + c * CHUNK, CHUNK)], sem)

    def w_drain(c, stage, sem):
        pltpu.make_async_copy(
            stage, out_hbm.at[pl.ds(row_base + c * CHUNK, CHUNK)], sem
        ).wait()

    g_start(0, stages[0], gsems[0])
    g_start(1, stages[1], gsems[1])

    def ring_step(h, carry):
        for k in range(4):
            c = h * 4 + k
            kn = (k + 2) % 4

            @pl.when(c < NCH)
            def _(c=c, k=k, kn=kn):
                @pl.when(c >= 2)
                def _():
                    w_drain(c - 2, stages[kn], wsems[kn])

                @pl.when(c + 2 < NCH)
                def _():
                    g_start(c + 2, stages[kn], gsems[kn])

                g_wait(c, stages[k], gsems[k])
                w_start(c, stages[k], wsems[k])

        return carry

    lax.fori_loop(0, (NCH + 3) // 4, ring_step, 0)

    w_drain(NCH - 2, stages[(NCH - 2) % 4], wsems[(NCH - 2) % 4])
    w_drain(NCH - 1, stages[(NCH - 1) % 4], wsems[(NCH - 1) % 4])


def kernel(edge_embedding, nbr_idx):
    table = edge_embedding.reshape(NT, F)
    nbr_flat = nbr_idx.reshape(NT)
    out = _gather_kernel(table, nbr_flat)
    return out.reshape(B, AT, K, NBR, F).transpose(0, 1, 3, 2, 4)


# speedup vs baseline: 6.4906x; 1.0142x over previous
import functools

import jax
import jax.numpy as jnp
from jax import lax
from jax.experimental import pallas as pl
from jax.experimental.pallas import tpu as pltpu
from jax.experimental.pallas import tpu_sc as plsc

B, AT, NBR, F = 8, 96, 16, 128
K = NBR - 1
NT = B * AT * NBR
NOUT = NT * K
NW = 32
ROWS_W = NOUT // NW
ATOMS_W = ROWS_W // (K * NBR)
CHUNK = 128
NCH = ROWS_W // CHUNK


@functools.partial(
    pl.kernel,
    mesh=plsc.VectorSubcoreMesh(core_axis_name="c", subcore_axis_name="s"),
    out_type=jax.ShapeDtypeStruct((NOUT, F), jnp.float32),
    compiler_params=pltpu.CompilerParams(needs_layout_passes=False),
    scratch_types=[
        pltpu.VMEM_SHARED((NT // 2, F), jnp.float32),
        pltpu.VMEM((ATOMS_W * NBR,), jnp.int32),
        pltpu.VMEM((ROWS_W,), jnp.int32),
        pltpu.VMEM((CHUNK, F), jnp.float32),
        pltpu.VMEM((CHUNK, F), jnp.float32),
        pltpu.VMEM((CHUNK, F), jnp.float32),
        pltpu.VMEM((CHUNK, F), jnp.float32),
        pltpu.SemaphoreType.DMA,
        pltpu.SemaphoreType.DMA,
        pltpu.SemaphoreType.DMA,
        pltpu.SemaphoreType.DMA,
        pltpu.SemaphoreType.DMA,
        pltpu.SemaphoreType.DMA,
        pltpu.SemaphoreType.DMA,
        pltpu.SemaphoreType.DMA,
        pltpu.SemaphoreType.DMA,
    ],
)
def _gather_kernel(table_hbm, nbr_hbm, out_hbm, slab, nbr_v, idx_v,
                   stage_0, stage_1, stage_2, stage_3,
                   gsem_0, gsem_1, gsem_2, gsem_3,
                   wsem_0, wsem_1, wsem_2, wsem_3, ssem):
    stages = [stage_0, stage_1, stage_2, stage_3]
    gsems = [gsem_0, gsem_1, gsem_2, gsem_3]
    wsems = [wsem_0, wsem_1, wsem_2, wsem_3]
    cid = lax.axis_index("c")
    sid = lax.axis_index("s")
    wid = cid * 16 + sid
    atom_base = wid * ATOMS_W
    row_base = wid * ROWS_W
    mol = atom_base // AT

    slab_rows = NT // 2 // 16
    slab_src = table_hbm.at[pl.ds(cid * (NT // 2) + sid * slab_rows, slab_rows)]
    slab_dst = slab.at[pl.ds(sid * slab_rows, slab_rows)]
    pltpu.async_copy(slab_src, slab_dst, ssem)
    pltpu.sync_copy(nbr_hbm.at[pl.ds(atom_base * NBR, ATOMS_W * NBR)], nbr_v)

    iota = lax.iota(jnp.int32, 16)
    kcols = [s - lax.shift_right_arithmetic(iota - (s + 1), 31) for s in range(K)]

    mol_loc = mol - cid * (B // 2)

    def build_atom(m, carry):
        base_vec = (nbr_v[pl.ds(m * NBR, NBR)] + mol_loc * AT) * NBR
        for s in range(K):
            idx_v[pl.ds((m * K + s) * NBR, NBR)] = base_vec + kcols[s]
        return carry

    lax.fori_loop(0, ATOMS_W, build_atom, 0)

    pltpu.make_async_copy(slab_src, slab_dst, ssem).wait()
    plsc.subcore_barrier()

    def g_start(c, stage, sem):
        pltpu.async_copy(slab.at[idx_v.at[pl.ds(c * CHUNK, CHUNK)]], stage, sem)

    def g_wait(c, stage, sem):
        pltpu.make_async_copy(
            slab.at[idx_v.at[pl.ds(c * CHUNK, CHUNK)]], stage, sem
        ).wait()

    def w_start(c, stage, sem):
        pltpu.async_copy(stage, out_hbm.at[pl.ds(row_base + c * CHUNK, CHUNK)], sem)

    def w_drain(c, stage, sem):
        pltpu.make_async_copy(
            stage, out_hbm.at[pl.ds(row_base + c * CHUNK, CHUNK)], sem
        ).wait()

    g_start(0, stages[0], gsems[0])
    g_start(1, stages[1], gsems[1])

    def ring_step(h, carry):
        for k in range(4):
            c = h * 4 + k
            kn = (k + 2) % 4

            @pl.when(c < NCH)
            def _(c=c, k=k, kn=kn):
                @pl.when(c >= 2)
                def _():
                    w_drain(c - 2, stages[kn], wsems[kn])

                @pl.when(c + 2 < NCH)
                def _():
                    g_start(c + 2, stages[kn], gsems[kn])

                g_wait(c, stages[k], gsems[k])
                w_start(c, stages[k], wsems[k])

        return carry

    lax.fori_loop(0, (NCH + 3) // 4, ring_step, 0)

    w_drain(NCH - 2, stages[(NCH - 2) % 4], wsems[(NCH - 2) % 4])
    w_drain(NCH - 1, stages[(NCH - 1) % 4], wsems[(NCH - 1) % 4])


def kernel(edge_embedding, nbr_idx):
    table = edge_embedding.reshape(NT, F)
    nbr_flat = nbr_idx.reshape(NT)
    out = _gather_kernel(table, nbr_flat)
    return out.reshape(B, AT, K, NBR, F).transpose(0, 1, 3, 2, 4)
